# Initial kernel scaffold; baseline (speedup 1.0000x reference)
import functools

import jax
import jax.numpy as jnp
from jax import lax
from jax.experimental import pallas as pl
from jax.experimental.pallas import tpu as pltpu
from jax.experimental.pallas import tpu_sc as plsc

N = 10000
E = 160000
D = 256

NC = 2
NS = 16
NW = NC * NS
LANES = 16

R = 320
NPAD = NW * R
HALF = NS * R
ACC_ROWS = 5376
EB = 2000
EBC = EB + 16
EPW_DEG = E // NS

BR = 512


def _rsqrt16(v):
    i = lax.bitcast_convert_type(v, jnp.int32)
    y = lax.bitcast_convert_type(
        jnp.int32(0x5F3759DF) - lax.shift_right_arithmetic(i, 1), jnp.float32)
    for _ in range(3):
        y = y * (1.5 - 0.5 * v * y * y)
    return y


def _deg_body(src_hbm, dst_hbm, ro_hbm, ri_hbm, s2_hbm,
              sidx, didx, ones, zbuf, dov, div, ov0, ov1, ov2,
              deg_out_sh, deg_in_sh):
    c = lax.axis_index("c")
    s = lax.axis_index("s")

    def fill_z(i, _):
        zbuf[pl.ds(i * 16, 16)] = jnp.zeros((16,), jnp.float32)
        return 0
    lax.fori_loop(0, 640 // 16, fill_z, 0)

    def fill_o(i, _):
        ones[pl.ds(i * 16, 16)] = jnp.ones((16,), jnp.float32)
        return 0
    lax.fori_loop(0, EPW_DEG // 16, fill_o, 0)

    pltpu.sync_copy(zbuf, deg_out_sh.at[pl.ds(s * 640, 640)])
    pltpu.sync_copy(zbuf, deg_in_sh.at[pl.ds(s * 640, 640)])
    plsc.subcore_barrier()

    e0 = s * EPW_DEG
    pltpu.sync_copy(src_hbm.at[pl.ds(e0, EPW_DEG)], sidx)
    pltpu.sync_copy(dst_hbm.at[pl.ds(e0, EPW_DEG)], didx)
    pltpu.sync_copy(ones, deg_out_sh.at[sidx], add=True)
    pltpu.sync_copy(ones, deg_in_sh.at[didx], add=True)
    plsc.subcore_barrier()

    base = c * HALF + s * R
    pltpu.sync_copy(deg_out_sh.at[pl.ds(base, R)], dov)
    pltpu.sync_copy(deg_in_sh.at[pl.ds(base, R)], div)

    def rd_loop(i, _):
        do = jnp.maximum(dov[pl.ds(i * 16, 16)], 1.0)
        di = jnp.maximum(div[pl.ds(i * 16, 16)], 1.0)
        ri = _rsqrt16(di)
        ov0[pl.ds(i * 16, 16)] = _rsqrt16(do)
        ov1[pl.ds(i * 16, 16)] = ri
        ov2[pl.ds(i * 16, 16)] = ri / di
        return 0
    lax.fori_loop(0, R // 16, rd_loop, 0)

    pltpu.sync_copy(ov0, ro_hbm.at[pl.ds(base, R)])
    pltpu.sync_copy(ov1, ri_hbm.at[pl.ds(base, R)])
    pltpu.sync_copy(ov2, s2_hbm.at[pl.ds(base, R)])


def _sc_degrees(src, dst):
    mesh = plsc.VectorSubcoreMesh(core_axis_name="c", subcore_axis_name="s",
                                  num_cores=NC, num_subcores=NS)
    f = pl.kernel(
        _deg_body,
        out_type=[jax.ShapeDtypeStruct((NPAD,), jnp.float32)] * 3,
        mesh=mesh,
        scratch_types=[
            pltpu.VMEM((EPW_DEG,), jnp.int32),
            pltpu.VMEM((EPW_DEG,), jnp.int32),
            pltpu.VMEM((EPW_DEG,), jnp.float32),
            pltpu.VMEM((640,), jnp.float32),
            pltpu.VMEM((R,), jnp.float32),
            pltpu.VMEM((R,), jnp.float32),
            pltpu.VMEM((R,), jnp.float32),
            pltpu.VMEM((R,), jnp.float32),
            pltpu.VMEM((R,), jnp.float32),
            pltpu.VMEM_SHARED((NPAD,), jnp.float32),
            pltpu.VMEM_SHARED((NPAD,), jnp.float32),
        ],
    )
    return f(src, dst)


def _max_body(hp_hbm, src_hbm, dst_hbm, mx_hbm,
              sbuf, dbuf, csrc, cldst, gidx, rowbuf, sem, acc):
    w = lax.axis_index("c") * NS + lax.axis_index("s")
    lo = w * R
    lanes = lax.iota(jnp.int32, 16)

    def zrow(i, _):
        for cc in range(16):
            acc[i, pl.ds(cc * 16, 16)] = jnp.zeros((16,), jnp.float32)
        return 0
    lax.fori_loop(0, R, zrow, 0)

    def zidx(i, _):
        csrc[pl.ds(i * 16, 16)] = jnp.zeros((16,), jnp.int32)
        cldst[pl.ds(i * 16, 16)] = jnp.zeros((16,), jnp.int32)
        return 0
    lax.fori_loop(0, EBC // 16, zidx, 0)

    def chunk(k, _):
        e0 = k * EB
        pltpu.sync_copy(src_hbm.at[pl.ds(e0, EB)], sbuf)
        pltpu.sync_copy(dst_hbm.at[pl.ds(e0, EB)], dbuf)

        def scan(i, m):
            r = dbuf[pl.ds(i * 16, 16)] - lo
            msk = (r >= 0) & (r < R)
            sv = sbuf[pl.ds(i * 16, 16)]
            plsc.store_compressed(csrc.at[pl.ds(m, 16)], sv, mask=msk)
            plsc.store_compressed(cldst.at[pl.ds(m, 16)], r, mask=msk)
            return m + jnp.sum(jnp.where(msk, 1, 0).astype(jnp.int32))
        m = lax.fori_loop(0, EB // 16, scan, jnp.int32(0))

        def gloop(g, _):
            valid = (g * 16 + lanes) < m
            gidx[...] = jnp.where(valid, csrc[pl.ds(g * 16, 16)], 0)
            pltpu.async_copy(hp_hbm.at[gidx], rowbuf, sem).wait()
            ldv = cldst[pl.ds(g * 16, 16)]
            for j in range(16):
                @pl.when(g * 16 + j < m)
                def _():
                    rj = jnp.sum(jnp.where(lanes == j, ldv, 0).astype(jnp.int32))
                    for cc in range(16):
                        a = acc[rj, pl.ds(cc * 16, 16)]
                        b = rowbuf[j, pl.ds(cc * 16, 16)]
                        acc[rj, pl.ds(cc * 16, 16)] = jnp.maximum(a, b)
            return 0
        lax.fori_loop(0, (m + 15) // 16, gloop, 0)
        return 0
    lax.fori_loop(0, E // EB, chunk, 0)

    pltpu.sync_copy(acc, mx_hbm.at[pl.ds(lo, R), :])


def _sc_segment_max(hp2, src, dst):
    mesh = plsc.VectorSubcoreMesh(core_axis_name="c", subcore_axis_name="s",
                                  num_cores=NC, num_subcores=NS)
    f = pl.kernel(
        _max_body,
        out_type=jax.ShapeDtypeStruct((NPAD, D), jnp.float32),
        mesh=mesh,
        scratch_types=[
            pltpu.VMEM((EB,), jnp.int32),
            pltpu.VMEM((EB,), jnp.int32),
            pltpu.VMEM((EBC,), jnp.int32),
            pltpu.VMEM((EBC,), jnp.int32),
            pltpu.VMEM((16,), jnp.int32),
            pltpu.VMEM((16, D), jnp.float32),
            pltpu.SemaphoreType.DMA,
            pltpu.VMEM((R, D), jnp.float32),
        ],
    )
    return f(hp2, src, dst)


def _sum_body(h2_hbm, src_hbm, dst_hbm, sm_hbm,
              sbuf, dbuf, csrc, cldst, cd2, gidx, rowbuf, zrow, sem, acc_sh):
    c = lax.axis_index("c")
    s = lax.axis_index("s")
    half_lo = c * HALF
    lanes = lax.iota(jnp.int32, 16)

    def zr(i, _):
        for cc in range(16):
            zrow[i, pl.ds(cc * 16, 16)] = jnp.zeros((16,), jnp.float32)
        return 0
    lax.fori_loop(0, 16, zr, 0)

    def zsh(i, _):
        pltpu.sync_copy(zrow, acc_sh.at[pl.ds(s * 336 + i * 16, 16), :])
        return 0
    lax.fori_loop(0, 336 // 16, zsh, 0)

    def zidx(i, _):
        csrc[pl.ds(i * 16, 16)] = jnp.zeros((16,), jnp.int32)
        return 0
    lax.fori_loop(0, EBC // 16, zidx, 0)
    plsc.subcore_barrier()

    def chunk(k, _):
        e0 = s * (E // NS) + k * EB
        pltpu.sync_copy(src_hbm.at[pl.ds(e0, EB)], sbuf)
        pltpu.sync_copy(dst_hbm.at[pl.ds(e0, EB)], dbuf)

        def scan(i, m):
            r = dbuf[pl.ds(i * 16, 16)] - half_lo
            msk = (r >= 0) & (r < HALF)
            sv = sbuf[pl.ds(i * 16, 16)]
            plsc.store_compressed(csrc.at[pl.ds(m, 16)], sv, mask=msk)
            plsc.store_compressed(cldst.at[pl.ds(m, 16)], r, mask=msk)
            return m + jnp.sum(jnp.where(msk, 1, 0).astype(jnp.int32))
        m = lax.fori_loop(0, EB // 16, scan, jnp.int32(0))
        ng = (m + 15) // 16

        def bld(g, _):
            lv = cldst[pl.ds(g * 16, 16)]
            valid = (g * 16 + lanes) < m
            cd2[g, :] = jnp.where(valid, lv, HALF + lanes)
            return 0
        lax.fori_loop(0, ng, bld, 0)

        def gloop(g, _):
            valid = (g * 16 + lanes) < m
            gidx[...] = jnp.where(valid, csrc[pl.ds(g * 16, 16)], 0)
            pltpu.async_copy(h2_hbm.at[gidx], rowbuf, sem).wait()
            pltpu.sync_copy(rowbuf, acc_sh.at[cd2.at[g]], add=True)
            return 0
        lax.fori_loop(0, ng, gloop, 0)
        return 0
    lax.fori_loop(0, E // NS // EB, chunk, 0)

    plsc.subcore_barrier()
    pltpu.sync_copy(acc_sh.at[pl.ds(s * R, R), :],
                    sm_hbm.at[pl.ds(c * HALF + s * R, R), :])


def _sc_segment_sum(h2, src, dst):
    mesh = plsc.VectorSubcoreMesh(core_axis_name="c", subcore_axis_name="s",
                                  num_cores=NC, num_subcores=NS)
    f = pl.kernel(
        _sum_body,
        out_type=jax.ShapeDtypeStruct((NPAD, D), jnp.float32),
        mesh=mesh,
        scratch_types=[
            pltpu.VMEM((EB,), jnp.int32),
            pltpu.VMEM((EB,), jnp.int32),
            pltpu.VMEM((EBC,), jnp.int32),
            pltpu.VMEM((EBC,), jnp.int32),
            pltpu.VMEM((EBC // 16, 16), jnp.int32),
            pltpu.VMEM((16,), jnp.int32),
            pltpu.VMEM((16, D), jnp.float32),
            pltpu.VMEM((16, D), jnp.float32),
            pltpu.SemaphoreType.DMA,
            pltpu.VMEM_SHARED((ACC_ROWS, D), jnp.float32),
        ],
    )
    return f(h2, src, dst)


def _b_body(x_ref, wp_ref, bp_ref, ro_ref, hp2_ref):
    hp = jnp.dot(x_ref[...], wp_ref[...], preferred_element_type=jnp.float32)
    hp = jnp.maximum(hp + bp_ref[...], 0.0)
    hp2_ref[...] = hp * ro_ref[...]


def _tc_conv1_pre(xp, W_pool, b_pool, ro):
    return pl.pallas_call(
        _b_body,
        grid=(NPAD // BR,),
        in_specs=[
            pl.BlockSpec((BR, D), lambda i: (i, 0)),
            pl.BlockSpec((D, D), lambda i: (0, 0)),
            pl.BlockSpec((1, D), lambda i: (0, 0)),
            pl.BlockSpec((BR, 1), lambda i: (i, 0)),
        ],
        out_specs=pl.BlockSpec((BR, D), lambda i: (i, 0)),
        out_shape=jax.ShapeDtypeStruct((NPAD, D), jnp.float32),
    )(xp, W_pool, b_pool.reshape(1, D), ro.reshape(NPAD, 1))


def _d_body(x_ref, mx_ref, ws_ref, wn_ref, b_ref, ri_ref, ro_ref, h_ref, h2_ref):
    rst = (jnp.dot(x_ref[...], ws_ref[...], preferred_element_type=jnp.float32)
           + ri_ref[...] * jnp.dot(mx_ref[...], wn_ref[...],
                                   preferred_element_type=jnp.float32)
           + b_ref[...])
    nrm = jnp.sqrt(jnp.sum(rst * rst, axis=1, keepdims=True))
    h = jnp.maximum(rst / jnp.maximum(nrm, 1e-12), 0.0)
    h_ref[...] = h
    h2_ref[...] = h * ro_ref[...]


def _tc_conv1_post(xp, mx, W_self1, W_neigh1, b1, ri, ro):
    return pl.pallas_call(
        _d_body,
        grid=(NPAD // BR,),
        in_specs=[
            pl.BlockSpec((BR, D), lambda i: (i, 0)),
            pl.BlockSpec((BR, D), lambda i: (i, 0)),
            pl.BlockSpec((D, D), lambda i: (0, 0)),
            pl.BlockSpec((D, D), lambda i: (0, 0)),
            pl.BlockSpec((1, D), lambda i: (0, 0)),
            pl.BlockSpec((BR, 1), lambda i: (i, 0)),
            pl.BlockSpec((BR, 1), lambda i: (i, 0)),
        ],
        out_specs=[pl.BlockSpec((BR, D), lambda i: (i, 0))] * 2,
        out_shape=[jax.ShapeDtypeStruct((NPAD, D), jnp.float32)] * 2,
    )(xp, mx, W_self1, W_neigh1, b1.reshape(1, D),
      ri.reshape(NPAD, 1), ro.reshape(NPAD, 1))


def _f_body(h_ref, sm_ref, ws_ref, wn_ref, b_ref, s2_ref, o_ref):
    o = (jnp.dot(h_ref[...], ws_ref[...], preferred_element_type=jnp.float32)
         + s2_ref[...] * jnp.dot(sm_ref[...], wn_ref[...],
                                 preferred_element_type=jnp.float32)
         + b_ref[...])
    o_ref[...] = jnp.maximum(o, 0.0)


def _tc_conv2(h, sm, W_self2, W_neigh2, b2, s2):
    return pl.pallas_call(
        _f_body,
        grid=(NPAD // BR,),
        in_specs=[
            pl.BlockSpec((BR, D), lambda i: (i, 0)),
            pl.BlockSpec((BR, D), lambda i: (i, 0)),
            pl.BlockSpec((D, D), lambda i: (0, 0)),
            pl.BlockSpec((D, D), lambda i: (0, 0)),
            pl.BlockSpec((1, D), lambda i: (0, 0)),
            pl.BlockSpec((BR, 1), lambda i: (i, 0)),
        ],
        out_specs=pl.BlockSpec((BR, D), lambda i: (i, 0)),
        out_shape=jax.ShapeDtypeStruct((NPAD, D), jnp.float32),
    )(h, sm, W_self2, W_neigh2, b2.reshape(1, D), s2.reshape(NPAD, 1))


def kernel(x, edge_index, edge_attr, W_pool, b_pool,
           W_self1, W_neigh1, b1, W_self2, W_neigh2, b2):
    del edge_attr
    src = edge_index[0].astype(jnp.int32)
    dst = edge_index[1].astype(jnp.int32)

    ro, ri, s2 = _sc_degrees(src, dst)

    xp = jnp.pad(x, ((0, NPAD - N), (0, 0)))
    hp2 = _tc_conv1_pre(xp, W_pool, b_pool, ro)
    mx = _sc_segment_max(hp2, src, dst)
    h, h2 = _tc_conv1_post(xp, mx, W_self1, W_neigh1, b1, ri, ro)
    sm = _sc_segment_sum(h2, src, dst)
    out = _tc_conv2(h, sm, W_self2, W_neigh2, b2, s2)
    return out[:N]


# speedup vs baseline: 1.6580x; 1.6580x over previous
import jax
import jax.numpy as jnp
from jax import lax
from jax.experimental import pallas as pl
from jax.experimental.pallas import tpu as pltpu
from jax.experimental.pallas import tpu_sc as plsc

N = 10000
E = 160000
D = 256

NC = 2
NS = 16
NW = NC * NS

R = 320
NPAD = NW * R
HALF = NS * R
EB = 2000
EBC = EB + 16
EPW_DEG = E // NS

BR = 512


def _pgather(x, idx):
    dn = lax.GatherDimensionNumbers(offset_dims=(), collapsed_slice_dims=(0,),
                                    start_index_map=(0,))
    return lax.gather(x, idx[:, None], dn, slice_sizes=(1,),
                      mode=lax.GatherScatterMode.PROMISE_IN_BOUNDS)


def _rsqrt16(v):
    i = lax.bitcast_convert_type(v, jnp.int32)
    y = lax.bitcast_convert_type(
        jnp.int32(0x5F3759DF) - lax.shift_right_arithmetic(i, 1), jnp.float32)
    for _ in range(3):
        y = y * (1.5 - 0.5 * v * y * y)
    return y


def _sc_mesh():
    return plsc.VectorSubcoreMesh(core_axis_name="c", subcore_axis_name="s",
                                  num_cores=NC, num_subcores=NS)


def _deg_body(src_hbm, dst_hbm, ro_hbm, ri_hbm, s2_hbm,
              sidx, didx, ones, zbuf, dov, div, ov0, ov1, ov2,
              deg_out_sh, deg_in_sh):
    c = lax.axis_index("c")
    s = lax.axis_index("s")

    def fill_z(i, _):
        zbuf[pl.ds(i * 16, 16)] = jnp.zeros((16,), jnp.float32)
        return 0
    lax.fori_loop(0, 640 // 16, fill_z, 0)

    def fill_o(i, _):
        ones[pl.ds(i * 16, 16)] = jnp.ones((16,), jnp.float32)
        return 0
    lax.fori_loop(0, EPW_DEG // 16, fill_o, 0)

    pltpu.sync_copy(zbuf, deg_out_sh.at[pl.ds(s * 640, 640)])
    pltpu.sync_copy(zbuf, deg_in_sh.at[pl.ds(s * 640, 640)])
    plsc.subcore_barrier()

    e0 = s * EPW_DEG
    pltpu.sync_copy(src_hbm.at[pl.ds(e0, EPW_DEG)], sidx)
    pltpu.sync_copy(dst_hbm.at[pl.ds(e0, EPW_DEG)], didx)
    pltpu.sync_copy(ones, deg_out_sh.at[sidx], add=True)
    pltpu.sync_copy(ones, deg_in_sh.at[didx], add=True)
    plsc.subcore_barrier()

    base = c * HALF + s * R
    pltpu.sync_copy(deg_out_sh.at[pl.ds(base, R)], dov)
    pltpu.sync_copy(deg_in_sh.at[pl.ds(base, R)], div)

    def rd_loop(i, _):
        do = jnp.maximum(dov[pl.ds(i * 16, 16)], 1.0)
        di = jnp.maximum(div[pl.ds(i * 16, 16)], 1.0)
        ri = _rsqrt16(di)
        ov0[pl.ds(i * 16, 16)] = _rsqrt16(do)
        ov1[pl.ds(i * 16, 16)] = ri
        ov2[pl.ds(i * 16, 16)] = ri / di
        return 0
    lax.fori_loop(0, R // 16, rd_loop, 0)

    pltpu.sync_copy(ov0, ro_hbm.at[pl.ds(base, R)])
    pltpu.sync_copy(ov1, ri_hbm.at[pl.ds(base, R)])
    pltpu.sync_copy(ov2, s2_hbm.at[pl.ds(base, R)])


def _sc_degrees(src, dst):
    f = pl.kernel(
        _deg_body,
        out_type=[jax.ShapeDtypeStruct((NPAD,), jnp.float32)] * 3,
        mesh=_sc_mesh(),
        scratch_types=[
            pltpu.VMEM((EPW_DEG,), jnp.int32),
            pltpu.VMEM((EPW_DEG,), jnp.int32),
            pltpu.VMEM((EPW_DEG,), jnp.float32),
            pltpu.VMEM((640,), jnp.float32),
            pltpu.VMEM((R,), jnp.float32),
            pltpu.VMEM((R,), jnp.float32),
            pltpu.VMEM((R,), jnp.float32),
            pltpu.VMEM((R,), jnp.float32),
            pltpu.VMEM((R,), jnp.float32),
            pltpu.VMEM_SHARED((NPAD,), jnp.float32),
            pltpu.VMEM_SHARED((NPAD,), jnp.float32),
        ],
    )
    return f(src, dst)


def _make_seg_body(is_max):
    def body(val_hbm, src_hbm, dst_hbm, out_hbm,
             sbuf, dbuf, cpk, gidx, rowbuf, sem, acc):
        w = lax.axis_index("c") * NS + lax.axis_index("s")
        lo = w * R
        lanes = lax.iota(jnp.int32, 16)
        steps = [(jnp.maximum(lanes - st, 0), lanes >= st) for st in (1, 2, 4, 8)]

        def zrow(i, _):
            for cc in range(16):
                acc[i, pl.ds(cc * 16, 16)] = jnp.zeros((16,), jnp.float32)
            return 0
        lax.fori_loop(0, R, zrow, 0)

        def chunk(k, _):
            e0 = k * EB
            pltpu.sync_copy(src_hbm.at[pl.ds(e0, EB)], sbuf)
            pltpu.sync_copy(dst_hbm.at[pl.ds(e0, EB)], dbuf)

            def scan(i, m):
                r = dbuf[pl.ds(i * 16, 16)] - lo
                msk = (r >= 0) & (r < R)
                sv = sbuf[pl.ds(i * 16, 16)]
                ps = jnp.where(msk, 1, 0)
                for iv, sm_ in steps:
                    ps = ps + jnp.where(sm_, _pgather(ps, iv), 0)
                cnt = ps[15]
                pk = jnp.where(msk, r * 16384 + sv, 0)

                def one():
                    t = pk
                    for iv, sm_ in steps:
                        t = t + jnp.where(sm_, _pgather(t, iv), 0)
                    cpk[pl.ds(m, 16)] = jnp.broadcast_to(t[15], (16,))
                    return m + 1

                def many():
                    pos = jnp.where(msk, ps - 1, 16 + lanes)
                    perm = lanes
                    for l in range(16):
                        perm = jnp.where(lanes == pos[l], l, perm)
                    cpk[pl.ds(m, 16)] = _pgather(pk, perm)
                    return m + cnt

                return lax.cond(cnt > 0,
                                lambda: lax.cond(cnt == 1, one, many),
                                lambda: m)
            m = lax.fori_loop(0, EB // 16, scan, jnp.int32(0))

            def gloop(g, _):
                pv = cpk[pl.ds(g * 16, 16)]
                valid = (g * 16 + lanes) < m
                gidx[...] = jnp.where(valid, pv & 16383, 0)
                pltpu.async_copy(val_hbm.at[gidx], rowbuf, sem).wait()
                ldv = jnp.right_shift(pv, 14)
                for j in range(16):
                    @pl.when(g * 16 + j < m)
                    def _():
                        rj = ldv[j]
                        for cc in range(16):
                            a = acc[rj, pl.ds(cc * 16, 16)]
                            b = rowbuf[j, pl.ds(cc * 16, 16)]
                            acc[rj, pl.ds(cc * 16, 16)] = (
                                jnp.maximum(a, b) if is_max else a + b)
                return 0
            lax.fori_loop(0, (m + 15) // 16, gloop, 0)
            return 0
        lax.fori_loop(0, E // EB, chunk, 0)

        pltpu.sync_copy(acc, out_hbm.at[pl.ds(lo, R), :])
    return body


def _sc_segment(vals, src, dst, is_max):
    f = pl.kernel(
        _make_seg_body(is_max),
        out_type=jax.ShapeDtypeStruct((NPAD, D), jnp.float32),
        mesh=_sc_mesh(),
        scratch_types=[
            pltpu.VMEM((EB,), jnp.int32),
            pltpu.VMEM((EB,), jnp.int32),
            pltpu.VMEM((EBC,), jnp.int32),
            pltpu.VMEM((16,), jnp.int32),
            pltpu.VMEM((16, D), jnp.float32),
            pltpu.SemaphoreType.DMA,
            pltpu.VMEM((R, D), jnp.float32),
        ],
    )
    return f(vals, src, dst)


def _b_body(x_ref, wp_ref, bp_ref, ro_ref, hp2_ref):
    hp = jnp.dot(x_ref[...], wp_ref[...], preferred_element_type=jnp.float32)
    hp = jnp.maximum(hp + bp_ref[...], 0.0)
    hp2_ref[...] = hp * ro_ref[...]


def _tc_conv1_pre(xp, W_pool, b_pool, ro):
    return pl.pallas_call(
        _b_body,
        grid=(NPAD // BR,),
        in_specs=[
            pl.BlockSpec((BR, D), lambda i: (i, 0)),
            pl.BlockSpec((D, D), lambda i: (0, 0)),
            pl.BlockSpec((1, D), lambda i: (0, 0)),
            pl.BlockSpec((BR, 1), lambda i: (i, 0)),
        ],
        out_specs=pl.BlockSpec((BR, D), lambda i: (i, 0)),
        out_shape=jax.ShapeDtypeStruct((NPAD, D), jnp.float32),
    )(xp, W_pool, b_pool.reshape(1, D), ro.reshape(NPAD, 1))


def _d_body(x_ref, mx_ref, ws_ref, wn_ref, b_ref, ri_ref, ro_ref, h_ref, h2_ref):
    rst = (jnp.dot(x_ref[...], ws_ref[...], preferred_element_type=jnp.float32)
           + ri_ref[...] * jnp.dot(mx_ref[...], wn_ref[...],
                                   preferred_element_type=jnp.float32)
           + b_ref[...])
    nrm = jnp.sqrt(jnp.sum(rst * rst, axis=1, keepdims=True))
    h = jnp.maximum(rst / jnp.maximum(nrm, 1e-12), 0.0)
    h_ref[...] = h
    h2_ref[...] = h * ro_ref[...]


def _tc_conv1_post(xp, mx, W_self1, W_neigh1, b1, ri, ro):
    return pl.pallas_call(
        _d_body,
        grid=(NPAD // BR,),
        in_specs=[
            pl.BlockSpec((BR, D), lambda i: (i, 0)),
            pl.BlockSpec((BR, D), lambda i: (i, 0)),
            pl.BlockSpec((D, D), lambda i: (0, 0)),
            pl.BlockSpec((D, D), lambda i: (0, 0)),
            pl.BlockSpec((1, D), lambda i: (0, 0)),
            pl.BlockSpec((BR, 1), lambda i: (i, 0)),
            pl.BlockSpec((BR, 1), lambda i: (i, 0)),
        ],
        out_specs=[pl.BlockSpec((BR, D), lambda i: (i, 0))] * 2,
        out_shape=[jax.ShapeDtypeStruct((NPAD, D), jnp.float32)] * 2,
    )(xp, mx, W_self1, W_neigh1, b1.reshape(1, D),
      ri.reshape(NPAD, 1), ro.reshape(NPAD, 1))


def _f_body(h_ref, sm_ref, ws_ref, wn_ref, b_ref, s2_ref, o_ref):
    o = (jnp.dot(h_ref[...], ws_ref[...], preferred_element_type=jnp.float32)
         + s2_ref[...] * jnp.dot(sm_ref[...], wn_ref[...],
                                 preferred_element_type=jnp.float32)
         + b_ref[...])
    o_ref[...] = jnp.maximum(o, 0.0)


def _tc_conv2(h, sm, W_self2, W_neigh2, b2, s2):
    return pl.pallas_call(
        _f_body,
        grid=(NPAD // BR,),
        in_specs=[
            pl.BlockSpec((BR, D), lambda i: (i, 0)),
            pl.BlockSpec((BR, D), lambda i: (i, 0)),
            pl.BlockSpec((D, D), lambda i: (0, 0)),
            pl.BlockSpec((D, D), lambda i: (0, 0)),
            pl.BlockSpec((1, D), lambda i: (0, 0)),
            pl.BlockSpec((BR, 1), lambda i: (i, 0)),
        ],
        out_specs=pl.BlockSpec((BR, D), lambda i: (i, 0)),
        out_shape=jax.ShapeDtypeStruct((NPAD, D), jnp.float32),
    )(h, sm, W_self2, W_neigh2, b2.reshape(1, D), s2.reshape(NPAD, 1))


def kernel(x, edge_index, edge_attr, W_pool, b_pool,
           W_self1, W_neigh1, b1, W_self2, W_neigh2, b2):
    del edge_attr
    src = edge_index[0].astype(jnp.int32)
    dst = edge_index[1].astype(jnp.int32)

    ro, ri, s2 = _sc_degrees(src, dst)

    xp = jnp.pad(x, ((0, NPAD - N), (0, 0)))
    hp2 = _tc_conv1_pre(xp, W_pool, b_pool, ro)
    mx = _sc_segment(hp2, src, dst, is_max=True)
    h, h2 = _tc_conv1_post(xp, mx, W_self1, W_neigh1, b1, ri, ro)
    sm = _sc_segment(h2, src, dst, is_max=False)
    out = _tc_conv2(h, sm, W_self2, W_neigh2, b2, s2)
    return out[:N]
